# baseline (speedup 1.0000x reference)
import functools

import jax
import jax.numpy as jnp
from jax.experimental import pallas as pl
from jax.experimental.pallas import tpu as pltpu

N_NODES = 10000
DEPTH = 50
UNIT = 32
HID = 128
T = 5000


def _mlp_head_body(z_ref, wm1_ref, bm1_ref, wm2_ref, bm2_ref, wm3_ref, bm3_ref, out_ref):
    z = z_ref[...]
    a = jnp.maximum(jnp.dot(z, wm1_ref[...], preferred_element_type=jnp.float32) + bm1_ref[...], 0.0)
    b = jnp.maximum(jnp.dot(a, wm2_ref[...], preferred_element_type=jnp.float32) + bm2_ref[...], 0.0)
    out_ref[...] = jnp.dot(b, wm3_ref[...], preferred_element_type=jnp.float32) + bm3_ref[...]


def _mlp_head(z, Wm1, bm1, Wm2, bm2, Wm3, bm3):
    t = z.shape[0]
    blk = 1000
    grid = t // blk
    return pl.pallas_call(
        _mlp_head_body,
        grid=(grid,),
        in_specs=[
            pl.BlockSpec((blk, z.shape[1]), lambda i: (i, 0)),
            pl.BlockSpec(Wm1.shape, lambda i: (0, 0)),
            pl.BlockSpec(bm1.shape, lambda i: (0,)),
            pl.BlockSpec(Wm2.shape, lambda i: (0, 0)),
            pl.BlockSpec(bm2.shape, lambda i: (0,)),
            pl.BlockSpec(Wm3.shape, lambda i: (0, 0)),
            pl.BlockSpec(bm3.shape, lambda i: (0,)),
        ],
        out_specs=pl.BlockSpec((blk, Wm3.shape[1]), lambda i: (i, 0)),
        out_shape=jax.ShapeDtypeStruct((t, Wm3.shape[1]), jnp.float32),
    )(z, Wm1, bm1, Wm2, bm2, Wm3, bm3)


def _layernorm(x, g, b, eps=1e-12):
    mu = jnp.mean(x, axis=-1, keepdims=True)
    var = jnp.mean((x - mu) ** 2, axis=-1, keepdims=True)
    return (x - mu) * jax.lax.rsqrt(var + eps) * g + b


def kernel(xpath_tags_seq, xpath_subs_seq, text_embeddings, ids, edge_index,
           tag_table, sub_table, W1, b1, W2, b2, ln_g, ln_b,
           Wg, bg, Wm1, bm1, Wm2, bm2, Wm3, bm3):
    tag_e = jnp.take(tag_table, xpath_tags_seq, axis=0)
    sub_e = jnp.take(sub_table, xpath_subs_seq, axis=0)
    xe = (tag_e + sub_e).reshape(xpath_tags_seq.shape[0], DEPTH * UNIT)
    xe = jax.nn.relu(xe @ W1 + b1)
    xe = xe @ W2 + b2
    xe = _layernorm(xe, ln_g, ln_b)
    src = edge_index[0]
    dst = edge_index[1]
    msg = jnp.take(xe, src, axis=0)
    agg = jax.ops.segment_sum(msg, dst, num_segments=N_NODES)
    deg = jax.ops.segment_sum(jnp.ones((msg.shape[0],), jnp.float32), dst, num_segments=N_NODES)
    agg = agg / jnp.clip(deg, 1.0, None)[:, None]
    h = jax.nn.relu(agg @ Wg + bg)
    tx = jnp.take(h, ids, axis=0)
    z = jnp.concatenate([text_embeddings, tx], axis=1)
    return _mlp_head(z, Wm1, bm1, Wm2, bm2, Wm3, bm3)


# speedup vs baseline: 9.1398x; 9.1398x over previous
import functools

import jax
import jax.numpy as jnp
from jax import lax
from jax.experimental import pallas as pl
from jax.experimental.pallas import tpu as pltpu
from jax.experimental.pallas import tpu_sc as plsc

N_NODES = 10000
N_EDGES = 320000
DEPTH = 50
UNIT = 32
HID = 128
T = 5000

NC = 2
NS = 16
NW = NC * NS

_MESH = plsc.VectorSubcoreMesh(core_axis_name="c", subcore_axis_name="s")

_A_CH = 800
_A_PAIRS = 520000
_A_NCHUNK = _A_PAIRS // _A_CH
_A_ITERS = 22


@functools.partial(
    pl.kernel,
    out_type=[
        jax.ShapeDtypeStruct((_A_PAIRS, UNIT), jnp.float32),
        jax.ShapeDtypeStruct((_A_PAIRS, UNIT), jnp.float32),
    ],
    mesh=_MESH,
    compiler_params=pltpu.CompilerParams(use_tc_tiling_on_sc=False),
    scratch_types=[
        [pltpu.VMEM((_A_CH,), jnp.int32)] * 2,
        [pltpu.VMEM((_A_CH,), jnp.int32)] * 2,
        [pltpu.VMEM((_A_CH, UNIT), jnp.float32)] * 2,
        [pltpu.VMEM((_A_CH, UNIT), jnp.float32)] * 2,
        pltpu.VMEM_SHARED((256, UNIT), jnp.float32),
        pltpu.VMEM_SHARED((1024, UNIT), jnp.float32),
        [pltpu.SemaphoreType.DMA] * 2,
        [pltpu.SemaphoreType.DMA] * 2,
    ],
)
def _embed_gather(tags_hbm, subs_hbm, tagtab_hbm, subtab_hbm,
                  outT_hbm, outS_hbm, idxT, idxS, rowT, rowS,
                  tagtab_v, subtab_v, semT, semS):
    wid = lax.axis_index("s") * NC + lax.axis_index("c")

    @pl.when(lax.axis_index("s") == 0)
    def _():
        pltpu.sync_copy(tagtab_hbm, tagtab_v)
        pltpu.sync_copy(subtab_hbm, subtab_v)

    plsc.subcore_barrier()

    def fire(j, b):
        base = j * _A_CH
        pltpu.sync_copy(tags_hbm.at[pl.ds(base, _A_CH)], idxT[b])
        pltpu.sync_copy(subs_hbm.at[pl.ds(base, _A_CH)], idxS[b])
        pltpu.async_copy(tagtab_v.at[idxT[b]], rowT[b], semT[b])
        pltpu.async_copy(subtab_v.at[idxS[b]], rowS[b], semS[b])

    def drain(j, b):
        base = j * _A_CH
        pltpu.make_async_copy(tagtab_v.at[pl.ds(0, _A_CH)], rowT[b],
                              semT[b]).wait()
        pltpu.sync_copy(rowT[b], outT_hbm.at[pl.ds(base, _A_CH)])
        pltpu.make_async_copy(subtab_v.at[pl.ds(0, _A_CH)], rowS[b],
                              semS[b]).wait()
        pltpu.sync_copy(rowS[b], outS_hbm.at[pl.ds(base, _A_CH)])

    fire(wid, 0)

    def body(i, carry):
        for b in range(2):
            k = i * 2 + b
            j_cur = k * NW + wid
            j_nxt = j_cur + NW

            @pl.when(j_nxt < _A_NCHUNK)
            def _():
                fire(j_nxt, 1 - b)

            @pl.when(j_cur < _A_NCHUNK)
            def _():
                drain(j_cur, b)

        return carry

    lax.fori_loop(0, _A_ITERS // 2, body, 0)


_D_CH = 400
_D_EPT = N_EDGES // NW
_D_NCH = _D_EPT // _D_CH
_DW = 16
_ROWS_PT = 624


@functools.partial(
    pl.kernel,
    out_type=jax.ShapeDtypeStruct((NC * N_NODES, _DW), jnp.float32),
    mesh=_MESH,
    compiler_params=pltpu.CompilerParams(use_tc_tiling_on_sc=False),
    scratch_types=[
        pltpu.VMEM((_D_CH,), jnp.int32),
        pltpu.VMEM((_D_CH, _DW), jnp.float32),
        pltpu.VMEM((_ROWS_PT, _DW), jnp.float32),
        pltpu.VMEM_SHARED((N_NODES, _DW), jnp.float32),
    ],
)
def _degree(dst_hbm, out_hbm, didx, ones_v, zero_v, deg_sh):
    c = lax.axis_index("c")
    s = lax.axis_index("s")
    wid = s * NC + c

    pat = jnp.where(lax.iota(jnp.int32, 16) == 0, 1.0, 0.0).astype(jnp.float32)
    zpat = jnp.zeros((16,), jnp.float32)

    def initrow(r, carry):
        ones_v[r, :] = pat
        return carry

    lax.fori_loop(0, _D_CH, initrow, 0)

    def zrow(r, carry):
        zero_v[r, :] = zpat
        return carry

    lax.fori_loop(0, _ROWS_PT, zrow, 0)
    pltpu.sync_copy(zero_v, deg_sh.at[pl.ds(s * _ROWS_PT, _ROWS_PT)])

    @pl.when(s == NS - 1)
    def _():
        pltpu.sync_copy(zero_v.at[pl.ds(0, 16)],
                        deg_sh.at[pl.ds(NS * _ROWS_PT, 16)])

    plsc.subcore_barrier()

    def body(i, carry):
        base = wid * _D_EPT + i * _D_CH
        pltpu.sync_copy(dst_hbm.at[pl.ds(base, _D_CH)], didx)
        pltpu.sync_copy(ones_v, deg_sh.at[didx], add=True)
        return carry

    lax.fori_loop(0, _D_NCH, body, 0)
    plsc.subcore_barrier()
    pltpu.sync_copy(deg_sh.at[pl.ds(s * _ROWS_PT, _ROWS_PT)],
                    out_hbm.at[pl.ds(c * N_NODES + s * _ROWS_PT, _ROWS_PT)])

    @pl.when(s == NS - 1)
    def _():
        pltpu.sync_copy(deg_sh.at[pl.ds(NS * _ROWS_PT, 16)],
                        out_hbm.at[pl.ds(c * N_NODES + NS * _ROWS_PT, 16)])


_C_CH = 400
_C_EPT = N_EDGES // NS
_C_NCH = _C_EPT // _C_CH
_HHID = HID // NC
_C_RPT = 624


@functools.partial(
    pl.kernel,
    out_type=jax.ShapeDtypeStruct((NC * N_NODES, _HHID), jnp.float32),
    mesh=_MESH,
    compiler_params=pltpu.CompilerParams(use_tc_tiling_on_sc=False),
    scratch_types=[
        [pltpu.VMEM((_C_CH,), jnp.int32)] * 2,
        [pltpu.VMEM((_C_CH,), jnp.int32)] * 2,
        [pltpu.VMEM((_C_CH, _HHID), jnp.float32)] * 2,
        pltpu.VMEM_SHARED((N_NODES, _HHID), jnp.float32),
        [pltpu.SemaphoreType.DMA] * 2,
    ],
)
def _edge_agg(src_hbm, dst_hbm, xe2_hbm, out_hbm,
              sidx, didx, msgs, agg_sh, sem):
    c = lax.axis_index("c")
    s = lax.axis_index("s")
    roff = c * N_NODES

    zpat = jnp.zeros((16,), jnp.float32)

    def zrow(r, carry):
        def zcol(k, carry2):
            msgs[0][r, pl.ds(k * 16, 16)] = zpat
            msgs[1][r, pl.ds(k * 16, 16)] = zpat
            return carry2

        lax.fori_loop(0, _HHID // 16, zcol, 0)
        return carry

    lax.fori_loop(0, _C_CH, zrow, 0)
    pltpu.sync_copy(msgs[0], agg_sh.at[pl.ds(s * _C_RPT, _C_CH)])
    pltpu.sync_copy(msgs[0].at[pl.ds(0, _C_RPT - _C_CH)],
                    agg_sh.at[pl.ds(s * _C_RPT + _C_CH, _C_RPT - _C_CH)])

    @pl.when(s == NS - 1)
    def _():
        pltpu.sync_copy(msgs[0].at[pl.ds(0, 16)],
                        agg_sh.at[pl.ds(NS * _C_RPT, 16)])

    plsc.subcore_barrier()

    def fire(chunk, b):
        base = s * _C_EPT + chunk * _C_CH
        pltpu.sync_copy(src_hbm.at[pl.ds(base, _C_CH)], sidx[b])
        pltpu.sync_copy(dst_hbm.at[pl.ds(base, _C_CH)], didx[b])

        def adj(k, carry2):
            sidx[b][pl.ds(k * 16, 16)] = sidx[b][pl.ds(k * 16, 16)] + roff
            return carry2

        lax.fori_loop(0, _C_CH // 16, adj, 0)
        pltpu.async_copy(xe2_hbm.at[sidx[b]], msgs[b], sem[b])

    def drain(b):
        pltpu.make_async_copy(xe2_hbm.at[pl.ds(0, _C_CH)], msgs[b],
                              sem[b]).wait()
        pltpu.sync_copy(msgs[b], agg_sh.at[didx[b]], add=True)

    fire(0, 0)

    def body(i, carry):
        for b in range(2):
            nxt = i * 2 + b + 1

            @pl.when(nxt < _C_NCH)
            def _():
                fire(nxt, 1 - b)

            drain(b)
        return carry

    lax.fori_loop(0, _C_NCH // 2, body, 0)
    plsc.subcore_barrier()
    pltpu.sync_copy(agg_sh.at[pl.ds(s * _C_RPT, _C_RPT)],
                    out_hbm.at[pl.ds(roff + s * _C_RPT, _C_RPT)])

    @pl.when(s == NS - 1)
    def _():
        pltpu.sync_copy(agg_sh.at[pl.ds(NS * _C_RPT, 16)],
                        out_hbm.at[pl.ds(roff + NS * _C_RPT, 16)])


_T_PAD = 5120
_T_PT = _T_PAD // NW


@functools.partial(
    pl.kernel,
    out_type=jax.ShapeDtypeStruct((_T_PAD, HID), jnp.float32),
    mesh=_MESH,
    compiler_params=pltpu.CompilerParams(use_tc_tiling_on_sc=False),
    scratch_types=[
        pltpu.VMEM((_T_PT,), jnp.int32),
        pltpu.VMEM((_T_PT, HID), jnp.float32),
        pltpu.SemaphoreType.DMA,
    ],
)
def _ids_gather(ids_hbm, h_hbm, out_hbm, idx, rows, sem):
    wid = lax.axis_index("s") * NC + lax.axis_index("c")
    base = wid * _T_PT
    pltpu.sync_copy(ids_hbm.at[pl.ds(base, _T_PT)], idx)
    pltpu.async_copy(h_hbm.at[idx], rows, sem).wait()
    pltpu.sync_copy(rows, out_hbm.at[pl.ds(base, _T_PT)])


_B_BLK = 400


_NSLAB = 13


def _xe_body(*refs):
    tag_refs = refs[:_NSLAB]
    sub_refs = refs[_NSLAB:2 * _NSLAB]
    W1_ref, b1_ref, W2_ref, b2_ref, g_ref, bb_ref, out_ref = refs[2 * _NSLAB:]
    xs = [(tag_refs[k][...] + sub_refs[k][...]).astype(jnp.bfloat16)
          for k in range(_NSLAB)]
    x = jnp.concatenate(xs, axis=1)
    a = jnp.maximum(
        jnp.dot(x, W1_ref[...], preferred_element_type=jnp.float32) + b1_ref[...], 0.0)
    a = a.astype(jnp.bfloat16)
    y = jnp.dot(a, W2_ref[...], preferred_element_type=jnp.float32) + b2_ref[...]
    mu = jnp.mean(y, axis=1, keepdims=True)
    var = jnp.mean((y - mu) ** 2, axis=1, keepdims=True)
    out_ref[...] = (y - mu) * lax.rsqrt(var + 1e-12) * g_ref[...] + bb_ref[...]


def _xe_mlp(tagE, subE, W1big, b1, W2, b2, ln_g, ln_b):
    grid = N_NODES // _B_BLK
    nb = N_NODES // _B_BLK
    slab_specs = [
        pl.BlockSpec((_B_BLK, HID), lambda i, k=k: (k * nb + i, 0))
        for k in range(_NSLAB)
    ]
    return pl.pallas_call(
        _xe_body,
        grid=(grid,),
        in_specs=slab_specs + slab_specs + [
            pl.BlockSpec(W1big.shape, lambda i: (0, 0)),
            pl.BlockSpec(b1.shape, lambda i: (0,)),
            pl.BlockSpec(W2.shape, lambda i: (0, 0)),
            pl.BlockSpec(b2.shape, lambda i: (0,)),
            pl.BlockSpec(ln_g.shape, lambda i: (0,)),
            pl.BlockSpec(ln_b.shape, lambda i: (0,)),
        ],
        out_specs=pl.BlockSpec((_B_BLK, HID), lambda i: (i, 0)),
        out_shape=jax.ShapeDtypeStruct((N_NODES, HID), jnp.float32),
    )(*([tagE] * _NSLAB + [subE] * _NSLAB
        + [W1big, b1, W2, b2, ln_g, ln_b]))


_H_BLK = 1000


def _h_body(aL_ref, aR_ref, d0_ref, d1_ref, WgT_ref, WgB_ref, bg_ref, out_ref):
    deg = jnp.clip(d0_ref[:, 0:1] + d1_ref[:, 0:1], 1.0, None)
    xL = aL_ref[...] / deg
    xR = aR_ref[...] / deg
    out_ref[...] = jnp.maximum(
        jnp.dot(xL, WgT_ref[...], preferred_element_type=jnp.float32)
        + jnp.dot(xR, WgB_ref[...], preferred_element_type=jnp.float32)
        + bg_ref[...], 0.0)


def _h_mlp(agg2, deg2, Wg, bg):
    grid = N_NODES // _H_BLK
    return pl.pallas_call(
        _h_body,
        grid=(grid,),
        in_specs=[
            pl.BlockSpec((_H_BLK, _HHID), lambda i: (i, 0)),
            pl.BlockSpec((_H_BLK, _HHID), lambda i: (i + N_NODES // _H_BLK, 0)),
            pl.BlockSpec((_H_BLK, _DW), lambda i: (i, 0)),
            pl.BlockSpec((_H_BLK, _DW), lambda i: (i + N_NODES // _H_BLK, 0)),
            pl.BlockSpec((_HHID, HID), lambda i: (0, 0)),
            pl.BlockSpec((_HHID, HID), lambda i: (0, 0)),
            pl.BlockSpec(bg.shape, lambda i: (0,)),
        ],
        out_specs=pl.BlockSpec((_H_BLK, HID), lambda i: (i, 0)),
        out_shape=jax.ShapeDtypeStruct((N_NODES, HID), jnp.float32),
    )(agg2, agg2, deg2, deg2, Wg[:_HHID], Wg[_HHID:], bg)


_F_BLK = 1000


def _head_body(tx_ref, te_ref, W1a_ref, W1b_ref, b1_ref, W2_ref, b2_ref,
               W3_ref, b3_ref, out_ref):
    a = jnp.maximum(
        jnp.dot(te_ref[...], W1a_ref[...], preferred_element_type=jnp.float32)
        + jnp.dot(tx_ref[...], W1b_ref[...], preferred_element_type=jnp.float32)
        + b1_ref[...], 0.0)
    b = jnp.maximum(
        jnp.dot(a, W2_ref[...], preferred_element_type=jnp.float32) + b2_ref[...], 0.0)
    out_ref[...] = jnp.dot(b, W3_ref[...], preferred_element_type=jnp.float32) + b3_ref[...]


def _head(tx, text_emb, Wm1a, Wm1b, bm1, Wm2, bm2, Wm3, bm3):
    grid = T // _F_BLK
    return pl.pallas_call(
        _head_body,
        grid=(grid,),
        in_specs=[
            pl.BlockSpec((_F_BLK, HID), lambda i: (i, 0)),
            pl.BlockSpec((_F_BLK, HID), lambda i: (i, 0)),
            pl.BlockSpec(Wm1a.shape, lambda i: (0, 0)),
            pl.BlockSpec(Wm1b.shape, lambda i: (0, 0)),
            pl.BlockSpec(bm1.shape, lambda i: (0,)),
            pl.BlockSpec(Wm2.shape, lambda i: (0, 0)),
            pl.BlockSpec(bm2.shape, lambda i: (0,)),
            pl.BlockSpec(Wm3.shape, lambda i: (0, 0)),
            pl.BlockSpec(bm3.shape, lambda i: (0,)),
        ],
        out_specs=pl.BlockSpec((_F_BLK, Wm3.shape[1]), lambda i: (i, 0)),
        out_shape=jax.ShapeDtypeStruct((T, Wm3.shape[1]), jnp.float32),
    )(tx, text_emb, Wm1a, Wm1b, bm1, Wm2, bm2, Wm3, bm3)


def kernel(xpath_tags_seq, xpath_subs_seq, text_embeddings, ids, edge_index,
           tag_table, sub_table, W1, b1, W2, b2, ln_g, ln_b,
           Wg, bg, Wm1, bm1, Wm2, bm2, Wm3, bm3):
    def slab_perm(a):
        main = a[:, :48].reshape(N_NODES, 12, 4).transpose(1, 0, 2).reshape(-1)
        tail = a[:, 46:50].reshape(-1)
        return jnp.concatenate([main, tail])

    tags_flat = slab_perm(xpath_tags_seq)
    subs_flat = slab_perm(xpath_subs_seq)
    tagE, subE = _embed_gather(tags_flat, subs_flat, tag_table, sub_table)
    tagE = tagE.reshape(_A_PAIRS * UNIT // HID, HID)
    subE = subE.reshape(_A_PAIRS * UNIT // HID, HID)
    W1big = jnp.concatenate(
        [W1[:1536], jnp.zeros((64, 4 * HID), W1.dtype), W1[1536:]], axis=0)
    xe = _xe_mlp(tagE, subE, W1big.astype(jnp.bfloat16), b1,
                 W2.astype(jnp.bfloat16), b2, ln_g, ln_b)

    src = edge_index[0]
    dst = edge_index[1]
    deg2 = _degree(dst)
    xe2 = jnp.concatenate([xe[:, :_HHID], xe[:, _HHID:]], axis=0)
    agg2 = _edge_agg(src, dst, xe2)
    h = _h_mlp(agg2, deg2, Wg, bg)

    ids_pad = jnp.concatenate([ids, jnp.zeros((_T_PAD - T,), jnp.int32)])
    tx = _ids_gather(ids_pad, h)[:T]
    return _head(tx, text_embeddings, Wm1[:HID], Wm1[HID:], bm1,
                 Wm2, bm2, Wm3, bm3)


# speedup vs baseline: 9.8116x; 1.0735x over previous
import functools

import jax
import jax.numpy as jnp
from jax import lax
from jax.experimental import pallas as pl
from jax.experimental.pallas import tpu as pltpu
from jax.experimental.pallas import tpu_sc as plsc

N_NODES = 10000
N_EDGES = 320000
DEPTH = 50
UNIT = 32
HID = 128
T = 5000

NC = 2
NS = 16
NW = NC * NS

_MESH = plsc.VectorSubcoreMesh(core_axis_name="c", subcore_axis_name="s")

_A_CH = 800
_A_NCHUNK = (N_NODES * DEPTH) // _A_CH
_A_ITERS = (_A_NCHUNK + NW - 1) // NW


@functools.partial(
    pl.kernel,
    out_type=[
        jax.ShapeDtypeStruct((N_NODES * DEPTH, UNIT), jnp.float32),
        jax.ShapeDtypeStruct((N_NODES * DEPTH, UNIT), jnp.float32),
    ],
    mesh=_MESH,
    compiler_params=pltpu.CompilerParams(use_tc_tiling_on_sc=False),
    scratch_types=[
        [pltpu.VMEM((_A_CH,), jnp.int32)] * 2,
        [pltpu.VMEM((_A_CH,), jnp.int32)] * 2,
        [pltpu.VMEM((_A_CH, UNIT), jnp.float32)] * 2,
        [pltpu.VMEM((_A_CH, UNIT), jnp.float32)] * 2,
        pltpu.VMEM_SHARED((256, UNIT), jnp.float32),
        pltpu.VMEM_SHARED((1024, UNIT), jnp.float32),
        [pltpu.SemaphoreType.DMA] * 2,
        [pltpu.SemaphoreType.DMA] * 2,
    ],
)
def _embed_gather(tags_hbm, subs_hbm, tagtab_hbm, subtab_hbm,
                  outT_hbm, outS_hbm, idxT, idxS, rowT, rowS,
                  tagtab_v, subtab_v, semT, semS):
    wid = lax.axis_index("s") * NC + lax.axis_index("c")

    @pl.when(lax.axis_index("s") == 0)
    def _():
        pltpu.sync_copy(tagtab_hbm, tagtab_v)
        pltpu.sync_copy(subtab_hbm, subtab_v)

    plsc.subcore_barrier()

    def fire(j, b):
        base = j * _A_CH
        pltpu.sync_copy(tags_hbm.at[pl.ds(base, _A_CH)], idxT[b])
        pltpu.sync_copy(subs_hbm.at[pl.ds(base, _A_CH)], idxS[b])
        pltpu.async_copy(tagtab_v.at[idxT[b]], rowT[b], semT[b])
        pltpu.async_copy(subtab_v.at[idxS[b]], rowS[b], semS[b])

    def drain(j, b):
        base = j * _A_CH
        pltpu.make_async_copy(tagtab_v.at[pl.ds(0, _A_CH)], rowT[b],
                              semT[b]).wait()
        pltpu.sync_copy(rowT[b], outT_hbm.at[pl.ds(base, _A_CH)])
        pltpu.make_async_copy(subtab_v.at[pl.ds(0, _A_CH)], rowS[b],
                              semS[b]).wait()
        pltpu.sync_copy(rowS[b], outS_hbm.at[pl.ds(base, _A_CH)])

    fire(wid, 0)

    def body(i, carry):
        for b in range(2):
            k = i * 2 + b
            j_cur = k * NW + wid
            j_nxt = j_cur + NW

            @pl.when(j_nxt < _A_NCHUNK)
            def _():
                fire(j_nxt, 1 - b)

            @pl.when(j_cur < _A_NCHUNK)
            def _():
                drain(j_cur, b)

        return carry

    lax.fori_loop(0, _A_ITERS // 2, body, 0)


_D_CH = 400
_D_EPT = N_EDGES // NW
_D_NCH = _D_EPT // _D_CH
_DW = 16
_ROWS_PT = 624


@functools.partial(
    pl.kernel,
    out_type=jax.ShapeDtypeStruct((NC * N_NODES, _DW), jnp.float32),
    mesh=_MESH,
    compiler_params=pltpu.CompilerParams(use_tc_tiling_on_sc=False),
    scratch_types=[
        pltpu.VMEM((_D_CH,), jnp.int32),
        pltpu.VMEM((_D_CH, _DW), jnp.float32),
        pltpu.VMEM((_ROWS_PT, _DW), jnp.float32),
        pltpu.VMEM_SHARED((N_NODES, _DW), jnp.float32),
    ],
)
def _degree(dst_hbm, out_hbm, didx, ones_v, zero_v, deg_sh):
    c = lax.axis_index("c")
    s = lax.axis_index("s")
    wid = s * NC + c

    pat = jnp.where(lax.iota(jnp.int32, 16) == 0, 1.0, 0.0).astype(jnp.float32)
    zpat = jnp.zeros((16,), jnp.float32)

    def initrow(r, carry):
        ones_v[r, :] = pat
        return carry

    lax.fori_loop(0, _D_CH, initrow, 0)

    def zrow(r, carry):
        zero_v[r, :] = zpat
        return carry

    lax.fori_loop(0, _ROWS_PT, zrow, 0)
    pltpu.sync_copy(zero_v, deg_sh.at[pl.ds(s * _ROWS_PT, _ROWS_PT)])

    @pl.when(s == NS - 1)
    def _():
        pltpu.sync_copy(zero_v.at[pl.ds(0, 16)],
                        deg_sh.at[pl.ds(NS * _ROWS_PT, 16)])

    plsc.subcore_barrier()

    def body(i, carry):
        base = wid * _D_EPT + i * _D_CH
        pltpu.sync_copy(dst_hbm.at[pl.ds(base, _D_CH)], didx)
        pltpu.sync_copy(ones_v, deg_sh.at[didx], add=True)
        return carry

    lax.fori_loop(0, _D_NCH, body, 0)
    plsc.subcore_barrier()
    pltpu.sync_copy(deg_sh.at[pl.ds(s * _ROWS_PT, _ROWS_PT)],
                    out_hbm.at[pl.ds(c * N_NODES + s * _ROWS_PT, _ROWS_PT)])

    @pl.when(s == NS - 1)
    def _():
        pltpu.sync_copy(deg_sh.at[pl.ds(NS * _ROWS_PT, 16)],
                        out_hbm.at[pl.ds(c * N_NODES + NS * _ROWS_PT, 16)])


_C_CH = 400
_C_EPT = N_EDGES // NS
_C_NCH = _C_EPT // _C_CH
_HHID = HID // NC
_C_RPT = 624


@functools.partial(
    pl.kernel,
    out_type=jax.ShapeDtypeStruct((NC * N_NODES, _HHID), jnp.float32),
    mesh=_MESH,
    compiler_params=pltpu.CompilerParams(use_tc_tiling_on_sc=False),
    scratch_types=[
        [pltpu.VMEM((_C_CH,), jnp.int32)] * 2,
        [pltpu.VMEM((_C_CH,), jnp.int32)] * 2,
        [pltpu.VMEM((_C_CH, _HHID), jnp.float32)] * 2,
        pltpu.VMEM_SHARED((N_NODES, _HHID), jnp.float32),
        [pltpu.SemaphoreType.DMA] * 2,
    ],
)
def _edge_agg(src_hbm, dst_hbm, xe2_hbm, out_hbm,
              sidx, didx, msgs, agg_sh, sem):
    c = lax.axis_index("c")
    s = lax.axis_index("s")
    roff = c * N_NODES

    zpat = jnp.zeros((16,), jnp.float32)

    def zrow(r, carry):
        def zcol(k, carry2):
            msgs[0][r, pl.ds(k * 16, 16)] = zpat
            msgs[1][r, pl.ds(k * 16, 16)] = zpat
            return carry2

        lax.fori_loop(0, _HHID // 16, zcol, 0)
        return carry

    lax.fori_loop(0, _C_CH, zrow, 0)
    pltpu.sync_copy(msgs[0], agg_sh.at[pl.ds(s * _C_RPT, _C_CH)])
    pltpu.sync_copy(msgs[0].at[pl.ds(0, _C_RPT - _C_CH)],
                    agg_sh.at[pl.ds(s * _C_RPT + _C_CH, _C_RPT - _C_CH)])

    @pl.when(s == NS - 1)
    def _():
        pltpu.sync_copy(msgs[0].at[pl.ds(0, 16)],
                        agg_sh.at[pl.ds(NS * _C_RPT, 16)])

    plsc.subcore_barrier()

    def fire(chunk, b):
        base = s * _C_EPT + chunk * _C_CH
        pltpu.sync_copy(src_hbm.at[pl.ds(base, _C_CH)], sidx[b])
        pltpu.sync_copy(dst_hbm.at[pl.ds(base, _C_CH)], didx[b])

        def adj(k, carry2):
            sidx[b][pl.ds(k * 16, 16)] = sidx[b][pl.ds(k * 16, 16)] + roff
            return carry2

        lax.fori_loop(0, _C_CH // 16, adj, 0)
        pltpu.async_copy(xe2_hbm.at[sidx[b]], msgs[b], sem[b])

    def drain(b):
        pltpu.make_async_copy(xe2_hbm.at[pl.ds(0, _C_CH)], msgs[b],
                              sem[b]).wait()
        pltpu.sync_copy(msgs[b], agg_sh.at[didx[b]], add=True)

    fire(0, 0)

    def body(i, carry):
        for b in range(2):
            nxt = i * 2 + b + 1

            @pl.when(nxt < _C_NCH)
            def _():
                fire(nxt, 1 - b)

            drain(b)
        return carry

    lax.fori_loop(0, _C_NCH // 2, body, 0)
    plsc.subcore_barrier()
    pltpu.sync_copy(agg_sh.at[pl.ds(s * _C_RPT, _C_RPT)],
                    out_hbm.at[pl.ds(roff + s * _C_RPT, _C_RPT)])

    @pl.when(s == NS - 1)
    def _():
        pltpu.sync_copy(agg_sh.at[pl.ds(NS * _C_RPT, 16)],
                        out_hbm.at[pl.ds(roff + NS * _C_RPT, 16)])


_T_PAD = 5120
_T_PT = _T_PAD // NW


@functools.partial(
    pl.kernel,
    out_type=jax.ShapeDtypeStruct((_T_PAD, HID), jnp.float32),
    mesh=_MESH,
    compiler_params=pltpu.CompilerParams(use_tc_tiling_on_sc=False),
    scratch_types=[
        pltpu.VMEM((_T_PT,), jnp.int32),
        pltpu.VMEM((_T_PT, HID), jnp.float32),
        pltpu.SemaphoreType.DMA,
    ],
)
def _ids_gather(ids_hbm, h_hbm, out_hbm, idx, rows, sem):
    wid = lax.axis_index("s") * NC + lax.axis_index("c")
    base = wid * _T_PT
    pltpu.sync_copy(ids_hbm.at[pl.ds(base, _T_PT)], idx)
    pltpu.async_copy(h_hbm.at[idx], rows, sem).wait()
    pltpu.sync_copy(rows, out_hbm.at[pl.ds(base, _T_PT)])


_B_BLK = 400


def _xe_body(tagE_ref, subE_ref, W1_ref, b1_ref, W2_ref, b2_ref,
             g_ref, bb_ref, out_ref):
    x = (tagE_ref[...] + subE_ref[...]).astype(jnp.bfloat16)
    a = jnp.maximum(
        jnp.dot(x, W1_ref[...], preferred_element_type=jnp.float32) + b1_ref[...], 0.0)
    a = a.astype(jnp.bfloat16)
    y = jnp.dot(a, W2_ref[...], preferred_element_type=jnp.float32) + b2_ref[...]
    mu = jnp.mean(y, axis=1, keepdims=True)
    var = jnp.mean((y - mu) ** 2, axis=1, keepdims=True)
    out_ref[...] = (y - mu) * lax.rsqrt(var + 1e-12) * g_ref[...] + bb_ref[...]


def _xe_mlp(tagE, subE, W1, b1, W2, b2, ln_g, ln_b):
    n = tagE.shape[0]
    grid = n // _B_BLK
    return pl.pallas_call(
        _xe_body,
        grid=(grid,),
        in_specs=[
            pl.BlockSpec((_B_BLK, DEPTH * UNIT), lambda i: (i, 0)),
            pl.BlockSpec((_B_BLK, DEPTH * UNIT), lambda i: (i, 0)),
            pl.BlockSpec(W1.shape, lambda i: (0, 0)),
            pl.BlockSpec(b1.shape, lambda i: (0,)),
            pl.BlockSpec(W2.shape, lambda i: (0, 0)),
            pl.BlockSpec(b2.shape, lambda i: (0,)),
            pl.BlockSpec(ln_g.shape, lambda i: (0,)),
            pl.BlockSpec(ln_b.shape, lambda i: (0,)),
        ],
        out_specs=pl.BlockSpec((_B_BLK, HID), lambda i: (i, 0)),
        out_shape=jax.ShapeDtypeStruct((n, HID), jnp.float32),
    )(tagE, subE, W1, b1, W2, b2, ln_g, ln_b)


_H_BLK = 1000


def _h_body(aL_ref, aR_ref, d0_ref, d1_ref, WgT_ref, WgB_ref, bg_ref, out_ref):
    deg = jnp.clip(d0_ref[:, 0:1] + d1_ref[:, 0:1], 1.0, None)
    xL = aL_ref[...] / deg
    xR = aR_ref[...] / deg
    out_ref[...] = jnp.maximum(
        jnp.dot(xL, WgT_ref[...], preferred_element_type=jnp.float32)
        + jnp.dot(xR, WgB_ref[...], preferred_element_type=jnp.float32)
        + bg_ref[...], 0.0)


def _h_mlp(agg2, deg2, Wg, bg):
    grid = N_NODES // _H_BLK
    return pl.pallas_call(
        _h_body,
        grid=(grid,),
        in_specs=[
            pl.BlockSpec((_H_BLK, _HHID), lambda i: (i, 0)),
            pl.BlockSpec((_H_BLK, _HHID), lambda i: (i + N_NODES // _H_BLK, 0)),
            pl.BlockSpec((_H_BLK, _DW), lambda i: (i, 0)),
            pl.BlockSpec((_H_BLK, _DW), lambda i: (i + N_NODES // _H_BLK, 0)),
            pl.BlockSpec((_HHID, HID), lambda i: (0, 0)),
            pl.BlockSpec((_HHID, HID), lambda i: (0, 0)),
            pl.BlockSpec(bg.shape, lambda i: (0,)),
        ],
        out_specs=pl.BlockSpec((_H_BLK, HID), lambda i: (i, 0)),
        out_shape=jax.ShapeDtypeStruct((N_NODES, HID), jnp.float32),
    )(agg2, agg2, deg2, deg2, Wg[:_HHID], Wg[_HHID:], bg)


_F_BLK = 1000


def _head_body(tx_ref, te_ref, W1a_ref, W1b_ref, b1_ref, W2_ref, b2_ref,
               W3_ref, b3_ref, out_ref):
    a = jnp.maximum(
        jnp.dot(te_ref[...], W1a_ref[...], preferred_element_type=jnp.float32)
        + jnp.dot(tx_ref[...], W1b_ref[...], preferred_element_type=jnp.float32)
        + b1_ref[...], 0.0)
    b = jnp.maximum(
        jnp.dot(a, W2_ref[...], preferred_element_type=jnp.float32) + b2_ref[...], 0.0)
    out_ref[...] = jnp.dot(b, W3_ref[...], preferred_element_type=jnp.float32) + b3_ref[...]


def _head(tx, text_emb, Wm1a, Wm1b, bm1, Wm2, bm2, Wm3, bm3):
    grid = T // _F_BLK
    return pl.pallas_call(
        _head_body,
        grid=(grid,),
        in_specs=[
            pl.BlockSpec((_F_BLK, HID), lambda i: (i, 0)),
            pl.BlockSpec((_F_BLK, HID), lambda i: (i, 0)),
            pl.BlockSpec(Wm1a.shape, lambda i: (0, 0)),
            pl.BlockSpec(Wm1b.shape, lambda i: (0, 0)),
            pl.BlockSpec(bm1.shape, lambda i: (0,)),
            pl.BlockSpec(Wm2.shape, lambda i: (0, 0)),
            pl.BlockSpec(bm2.shape, lambda i: (0,)),
            pl.BlockSpec(Wm3.shape, lambda i: (0, 0)),
            pl.BlockSpec(bm3.shape, lambda i: (0,)),
        ],
        out_specs=pl.BlockSpec((_F_BLK, Wm3.shape[1]), lambda i: (i, 0)),
        out_shape=jax.ShapeDtypeStruct((T, Wm3.shape[1]), jnp.float32),
    )(tx, text_emb, Wm1a, Wm1b, bm1, Wm2, bm2, Wm3, bm3)


def kernel(xpath_tags_seq, xpath_subs_seq, text_embeddings, ids, edge_index,
           tag_table, sub_table, W1, b1, W2, b2, ln_g, ln_b,
           Wg, bg, Wm1, bm1, Wm2, bm2, Wm3, bm3):
    tags_flat = xpath_tags_seq.reshape(-1)
    subs_flat = xpath_subs_seq.reshape(-1)
    tagE, subE = _embed_gather(tags_flat, subs_flat, tag_table, sub_table)
    tagE = tagE.reshape(N_NODES, DEPTH * UNIT)
    subE = subE.reshape(N_NODES, DEPTH * UNIT)
    xe = _xe_mlp(tagE, subE, W1.astype(jnp.bfloat16), b1,
                 W2.astype(jnp.bfloat16), b2, ln_g, ln_b)

    src = edge_index[0]
    dst = edge_index[1]
    deg2 = _degree(dst)
    xe2 = jnp.concatenate([xe[:, :_HHID], xe[:, _HHID:]], axis=0)
    agg2 = _edge_agg(src, dst, xe2)
    h = _h_mlp(agg2, deg2, Wg, bg)

    ids_pad = jnp.concatenate([ids, jnp.zeros((_T_PAD - T,), jnp.int32)])
    tx = _ids_gather(ids_pad, h)[:T]
    return _head(tx, text_embeddings, Wm1[:HID], Wm1[HID:], bm1,
                 Wm2, bm2, Wm3, bm3)
